# baseline (speedup 1.0000x reference)
import functools

import jax
import jax.numpy as jnp
from jax import lax
from jax.experimental import pallas as pl
from jax.experimental.pallas import tpu as pltpu
from jax.experimental.pallas import tpu_sc as plsc

N = 10000
E = 320000
H = 128

NC = 2
NS = 16
NW = NC * NS

NPAD = 10240
RPS = NPAD // NS
C = 80
EPW = E // NW
CPW = EPW // C

_MESH = dict(core_axis_name="c", subcore_axis_name="s", num_cores=NC,
             num_subcores=NS)


def _tc_transform(h_pad, Wsb, bsb):
    blk = 1024

    def body(h_ref, w_ref, b_ref, out_ref):
        out_ref[0] = (
            jnp.dot(h_ref[...], w_ref[0], preferred_element_type=jnp.float32)
            + b_ref[0]
        )

    return pl.pallas_call(
        body,
        grid=(2, NPAD // blk),
        in_specs=[
            pl.BlockSpec((blk, H), lambda j, i: (i, 0)),
            pl.BlockSpec((1, H, H), lambda j, i: (j, 0, 0)),
            pl.BlockSpec((1, 1, H), lambda j, i: (j, 0, 0)),
        ],
        out_specs=pl.BlockSpec((1, blk, H), lambda j, i: (j, i, 0)),
        out_shape=jax.ShapeDtypeStruct((2, NPAD, H), jnp.float32),
    )(h_pad, Wsb, bsb)


def _sc_gather(T, src2, dstp2, ne, cc):
    mesh = plsc.VectorSubcoreMesh(**_MESH)
    HW = H // 2
    epw = ne // NW
    cpw = epw // cc

    @functools.partial(
        pl.kernel,
        out_type=jax.ShapeDtypeStruct((ne, H), jnp.int32),
        mesh=mesh,
        scratch_types=[
            pltpu.VMEM((cpw, cc), jnp.int32),
            pltpu.VMEM((cpw, cc), jnp.int32),
            pltpu.VMEM((2, cc, H), jnp.int32),
            pltpu.VMEM((2, cc, H), jnp.int32),
            pltpu.SemaphoreType.DMA((2,)),
            pltpu.SemaphoreType.DMA((2,)),
            pltpu.SemaphoreType.DMA((2,)),
            pltpu.SemaphoreType.DMA((2,)),
        ],
    )
    def k(t_hbm, s_hbm, d_hbm, gs_hbm, si_v, di_v, bufs, bufd,
          gss, gsd, wss, wsd):
        wid = lax.axis_index("s") * NC + lax.axis_index("c")
        pltpu.sync_copy(s_hbm.at[wid], si_v)
        pltpu.sync_copy(d_hbm.at[wid], di_v)
        ebase = wid * epw

        def gather_start(i, b):
            pltpu.async_copy(t_hbm.at[si_v.at[i]], bufs.at[b], gss.at[b])
            pltpu.async_copy(t_hbm.at[di_v.at[i]], bufd.at[b], gsd.at[b])

        def gather_wait(b):
            pltpu.make_async_copy(t_hbm.at[si_v.at[0]], bufs.at[b], gss.at[b]).wait()
            pltpu.make_async_copy(t_hbm.at[di_v.at[0]], bufd.at[b], gsd.at[b]).wait()

        def wb_start(i, b):
            off = ebase + i * cc
            pltpu.async_copy(bufs.at[b], gs_hbm.at[pl.ds(off, cc)], wss.at[b])

        def wb_wait(b):
            pltpu.make_async_copy(bufs.at[b], gs_hbm.at[pl.ds(ebase, cc)],
                                  wss.at[b]).wait()

        gather_start(0, 0)

        def body(i, carry):
            b = lax.rem(i, 2)
            nb = 1 - b
            gather_wait(b)

            @plsc.parallel_loop(0, cc, step=1, unroll=8)
            def asm(r):
                for kq in range(HW // 16):
                    col = HW + 16 * kq
                    bufs[b, r, pl.ds(col, 16)] = bufd[b, r, pl.ds(col, 16)]

            wb_start(i, b)

            @pl.when(i + 1 < cpw)
            def _():
                @pl.when(i >= 1)
                def _():
                    wb_wait(nb)

                gather_start(i + 1, nb)

            return carry

        lax.fori_loop(0, cpw, body, 0)
        wb_wait(0)
        wb_wait(1)

    return k(T, src2, dstp2)


def _tc_edge(e, g_packed, prev, nblk_off, W_e, W_out, b_out, gamma_e, beta_e):
    blk = 8000
    HW = H // 2
    grid = g_packed.shape[0] // blk

    def body(e_ref, gp_ref, we_ref, wo_ref, bo_ref, g_ref, b_ref, *rest):
        out_ref = rest[-1]
        M16 = jnp.full((), -65536, jnp.int32)
        ev = e_ref[...]
        gp = gp_ref[...]
        ws = gp[:, :HW]
        wd = gp[:, HW:]
        lo = (lax.bitcast_convert_type(lax.shift_left(ws, 16), jnp.float32)
              + lax.bitcast_convert_type(lax.shift_left(wd, 16), jnp.float32))
        hi = (lax.bitcast_convert_type(ws & M16, jnp.float32)
              + lax.bitcast_convert_type(wd & M16, jnp.float32))
        z = jnp.concatenate([lo, hi], axis=1) + jnp.dot(
            ev.astype(jnp.bfloat16), we_ref[...],
            preferred_element_type=jnp.float32,
        )
        z = z * jax.nn.sigmoid(z)
        en = ev + jnp.dot(z.astype(jnp.bfloat16), wo_ref[...],
                          preferred_element_type=jnp.float32) + bo_ref[...]
        m = jnp.mean(en, axis=-1, keepdims=True)
        v = jnp.mean((en - m) ** 2, axis=-1, keepdims=True)
        out_ref[...] = (en - m) * lax.rsqrt(v + 1e-5) * g_ref[...] + b_ref[...]

    full = lambda i: (0, 0)
    in_specs = [
        pl.BlockSpec((blk, H), lambda i: (i + nblk_off, 0)),
        pl.BlockSpec((blk, H), lambda i: (i, 0)),
        pl.BlockSpec((H, H), full),
        pl.BlockSpec((H, H), full),
        pl.BlockSpec((1, H), full),
        pl.BlockSpec((1, H), full),
        pl.BlockSpec((1, H), full),
    ]
    args = [e, g_packed, W_e, W_out, b_out, gamma_e, beta_e]
    aliases = {}
    if prev is not None:
        in_specs.append(pl.BlockSpec(memory_space=pl.ANY))
        args.append(prev)
        aliases = {7: 0}
    return pl.pallas_call(
        body,
        grid=(grid,),
        in_specs=in_specs,
        out_specs=pl.BlockSpec((blk, H), lambda i: (i + nblk_off, 0)),
        out_shape=jax.ShapeDtypeStruct((E, H), jnp.float32),
        input_output_aliases=aliases,
    )(*args)


def _sc_scatter(e_new, dst2, zrows):
    mesh = plsc.VectorSubcoreMesh(**_MESH)

    @functools.partial(
        pl.kernel,
        out_type=jax.ShapeDtypeStruct((NC, NPAD, H), jnp.float32),
        mesh=mesh,
        scratch_types=[
            pltpu.VMEM((CPW, C), jnp.int32),
            pltpu.VMEM((2, C, H), jnp.float32),
            pltpu.VMEM_SHARED((NPAD, H), jnp.float32),
            pltpu.SemaphoreType.DMA((2,)),
        ],
    )
    def k(e_hbm, d_hbm, z_hbm, out_hbm, di_v, buf, acc, lsem):
        cid = lax.axis_index("c")
        sid = lax.axis_index("s")
        wid = sid * NC + cid
        row0 = sid * RPS
        ebase = wid * EPW

        def load_start(i, b):
            pltpu.async_copy(e_hbm.at[pl.ds(ebase + i * C, C)], buf.at[b],
                             lsem.at[b])

        def load_wait(b):
            pltpu.make_async_copy(e_hbm.at[pl.ds(ebase, C)], buf.at[b],
                                  lsem.at[b]).wait()

        load_start(0, 0)
        pltpu.sync_copy(z_hbm.at[pl.ds(row0, RPS)], acc.at[pl.ds(row0, RPS)])
        pltpu.sync_copy(d_hbm.at[wid], di_v)
        plsc.subcore_barrier()

        def body(i, carry):
            b = lax.rem(i, 2)
            load_wait(b)

            @pl.when(i + 1 < CPW)
            def _():
                load_start(i + 1, 1 - b)

            pltpu.sync_copy(buf.at[b], acc.at[di_v.at[i]], add=True)
            return carry

        lax.fori_loop(0, CPW, body, 0)
        plsc.subcore_barrier()
        pltpu.sync_copy(acc.at[pl.ds(row0, RPS)], out_hbm.at[cid, pl.ds(row0, RPS)])

    return k(e_new, dst2, zrows)


def _tc_node(h_pad, P, W_n1, b_n1, W_n2, b_n2, gamma_n, beta_n):
    blk = 1024
    full = lambda i: (0, 0)

    def body(h_ref, p0_ref, p1_ref, w1_ref, b1_ref, w2_ref, b2_ref, g_ref, b_ref,
             out_ref):
        hv = h_ref[...]
        agg = p0_ref[0] + p1_ref[0]
        w1 = w1_ref[...]
        x = (
            jnp.dot(hv, w1[:H], preferred_element_type=jnp.float32)
            + jnp.dot(agg, w1[H:], preferred_element_type=jnp.float32)
            + b1_ref[...]
        )
        x = x * jax.nn.sigmoid(x)
        hn = hv + jnp.dot(x, w2_ref[...], preferred_element_type=jnp.float32) + b2_ref[...]
        m = jnp.mean(hn, axis=-1, keepdims=True)
        v = jnp.mean((hn - m) ** 2, axis=-1, keepdims=True)
        out_ref[...] = (hn - m) * lax.rsqrt(v + 1e-5) * g_ref[...] + b_ref[...]

    return pl.pallas_call(
        body,
        grid=(NPAD // blk,),
        in_specs=[
            pl.BlockSpec((blk, H), lambda i: (i, 0)),
            pl.BlockSpec((1, blk, H), lambda i: (0, i, 0)),
            pl.BlockSpec((1, blk, H), lambda i: (1, i, 0)),
            pl.BlockSpec((2 * H, H), full),
            pl.BlockSpec((1, H), full),
            pl.BlockSpec((H, H), full),
            pl.BlockSpec((1, H), full),
            pl.BlockSpec((1, H), full),
            pl.BlockSpec((1, H), full),
        ],
        out_specs=pl.BlockSpec((blk, H), lambda i: (i, 0)),
        out_shape=jax.ShapeDtypeStruct((NPAD, H), jnp.float32),
    )(h_pad, P, P, W_n1, b_n1, W_n2, b_n2, gamma_n, beta_n)


def kernel(h, e, edge_index, W_src, b_src, W_dst, W_e, W_out, b_out, W_n1, b_n1,
           W_n2, b_n2, gamma_e, beta_e, gamma_n, beta_n):
    h_pad = jnp.zeros((NPAD, H), jnp.float32).at[:N].set(h)
    Wsb = jnp.stack([W_src, W_dst])
    bsb = jnp.stack([b_src, jnp.zeros_like(b_src)]).reshape(2, 1, H)

    T3 = _tc_transform(h_pad, Wsb, bsb)
    u = lax.bitcast_convert_type(T3.astype(jnp.bfloat16), jnp.uint16).astype(
        jnp.uint32)
    packed = lax.bitcast_convert_type(
        (u[:, :, H // 2:] << 16) | u[:, :, : H // 2], jnp.int32)
    T = jnp.concatenate([packed[0], packed[1]], axis=1)

    E2 = E // 2
    C2 = 40
    src = edge_index[0]
    dst = edge_index[1]
    g0 = _sc_gather(T, src[:E2].reshape(NW, -1, C2),
                    dst[:E2].reshape(NW, -1, C2), E2, C2)
    g1 = _sc_gather(T, src[E2:].reshape(NW, -1, C2),
                    dst[E2:].reshape(NW, -1, C2), E2, C2)
    web = W_e.astype(jnp.bfloat16)
    wob = W_out.astype(jnp.bfloat16)
    en0 = _tc_edge(e, g0, None, 0, web, wob, b_out.reshape(1, H),
                   gamma_e.reshape(1, H), beta_e.reshape(1, H))
    e_new = _tc_edge(e, g1, en0, E2 // 8000, web, wob, b_out.reshape(1, H),
                     gamma_e.reshape(1, H), beta_e.reshape(1, H))

    dst2 = edge_index[1].reshape(NW, CPW, C)
    zrows = jnp.zeros((NPAD, H), jnp.float32)
    P = _sc_scatter(e_new, dst2, zrows)

    h_new_pad = _tc_node(h_pad, P, W_n1, b_n1.reshape(1, H), W_n2,
                         b_n2.reshape(1, H), gamma_n.reshape(1, H),
                         beta_n.reshape(1, H))
    return h_new_pad[:N], e_new


# speedup vs baseline: 1.1202x; 1.1202x over previous
import functools

import jax
import jax.numpy as jnp
from jax import lax
from jax.experimental import pallas as pl
from jax.experimental.pallas import tpu as pltpu
from jax.experimental.pallas import tpu_sc as plsc

N = 10000
E = 320000
H = 128

NC = 2
NS = 16
NW = NC * NS

NPAD = 10240
RPS = NPAD // NS
C = 80
EPW = E // NW
CPW = EPW // C

_MESH = dict(core_axis_name="c", subcore_axis_name="s", num_cores=NC,
             num_subcores=NS)


def _tc_transform(h_pad, Wsb, bsb):
    blk = 1024

    def body(h_ref, w_ref, b_ref, out_ref):
        out_ref[0] = (
            jnp.dot(h_ref[...], w_ref[0], preferred_element_type=jnp.float32)
            + b_ref[0]
        )

    return pl.pallas_call(
        body,
        grid=(2, NPAD // blk),
        in_specs=[
            pl.BlockSpec((blk, H), lambda j, i: (i, 0)),
            pl.BlockSpec((1, H, H), lambda j, i: (j, 0, 0)),
            pl.BlockSpec((1, 1, H), lambda j, i: (j, 0, 0)),
        ],
        out_specs=pl.BlockSpec((1, blk, H), lambda j, i: (j, i, 0)),
        out_shape=jax.ShapeDtypeStruct((2, NPAD, H), jnp.float32),
    )(h_pad, Wsb, bsb)


def _sc_gather(T, src2, dstp2, ne, cc):
    mesh = plsc.VectorSubcoreMesh(**_MESH)
    HW = H // 2
    epw = ne // NW
    cpw = epw // cc

    @functools.partial(
        pl.kernel,
        out_type=jax.ShapeDtypeStruct((ne, H), jnp.int32),
        mesh=mesh,
        scratch_types=[
            pltpu.VMEM((cpw, cc), jnp.int32),
            pltpu.VMEM((cpw, cc), jnp.int32),
            pltpu.VMEM((2, cc, H), jnp.int32),
            pltpu.VMEM((2, cc, H), jnp.int32),
            pltpu.SemaphoreType.DMA((2,)),
            pltpu.SemaphoreType.DMA((2,)),
            pltpu.SemaphoreType.DMA((2,)),
            pltpu.SemaphoreType.DMA((2,)),
        ],
    )
    def k(t_hbm, s_hbm, d_hbm, gs_hbm, si_v, di_v, bufs, bufd,
          gss, gsd, wss, wsd):
        wid = lax.axis_index("s") * NC + lax.axis_index("c")
        pltpu.sync_copy(s_hbm.at[wid], si_v)
        pltpu.sync_copy(d_hbm.at[wid], di_v)
        ebase = wid * epw

        def gather_start(i, b):
            pltpu.async_copy(t_hbm.at[si_v.at[i]], bufs.at[b], gss.at[b])
            pltpu.async_copy(t_hbm.at[di_v.at[i]], bufd.at[b], gsd.at[b])

        def gather_wait(b):
            pltpu.make_async_copy(t_hbm.at[si_v.at[0]], bufs.at[b], gss.at[b]).wait()
            pltpu.make_async_copy(t_hbm.at[di_v.at[0]], bufd.at[b], gsd.at[b]).wait()

        def wb_start(i, b):
            off = ebase + i * cc
            pltpu.async_copy(bufs.at[b], gs_hbm.at[pl.ds(off, cc)], wss.at[b])

        def wb_wait(b):
            pltpu.make_async_copy(bufs.at[b], gs_hbm.at[pl.ds(ebase, cc)],
                                  wss.at[b]).wait()

        gather_start(0, 0)

        def body(i, carry):
            b = lax.rem(i, 2)
            nb = 1 - b
            gather_wait(b)

            @plsc.parallel_loop(0, cc, step=1, unroll=8)
            def asm(r):
                for kq in range(HW // 16):
                    col = HW + 16 * kq
                    bufs[b, r, pl.ds(col, 16)] = bufd[b, r, pl.ds(col, 16)]

            wb_start(i, b)

            @pl.when(i + 1 < cpw)
            def _():
                @pl.when(i >= 1)
                def _():
                    wb_wait(nb)

                gather_start(i + 1, nb)

            return carry

        lax.fori_loop(0, cpw, body, 0)
        wb_wait(0)
        wb_wait(1)

    return k(T, src2, dstp2)


def _tc_edge(e, g_packed, prev, nblk_off, W_e, W_out, b_out, gamma_e, beta_e):
    blk = 8000
    HW = H // 2
    grid = g_packed.shape[0] // blk

    def body(e_ref, gp_ref, we_ref, wo_ref, bo_ref, g_ref, b_ref, *rest):
        out_ref = rest[-1]
        M16 = jnp.full((), -65536, jnp.int32)
        ev = e_ref[...]
        gp = gp_ref[...]
        ws = gp[:, :HW]
        wd = gp[:, HW:]
        lo = (lax.bitcast_convert_type(lax.shift_left(ws, 16), jnp.float32)
              + lax.bitcast_convert_type(lax.shift_left(wd, 16), jnp.float32))
        hi = (lax.bitcast_convert_type(ws & M16, jnp.float32)
              + lax.bitcast_convert_type(wd & M16, jnp.float32))
        z = jnp.concatenate([lo, hi], axis=1) + jnp.dot(
            ev.astype(jnp.bfloat16), we_ref[...],
            preferred_element_type=jnp.float32,
        )
        z = z * jax.nn.sigmoid(z)
        en = ev + jnp.dot(z.astype(jnp.bfloat16), wo_ref[...],
                          preferred_element_type=jnp.float32) + bo_ref[...]
        m = jnp.mean(en, axis=-1, keepdims=True)
        v = jnp.mean((en - m) ** 2, axis=-1, keepdims=True)
        out_ref[...] = (en - m) * lax.rsqrt(v + 1e-5) * g_ref[...] + b_ref[...]

    full = lambda i: (0, 0)
    in_specs = [
        pl.BlockSpec((blk, H), lambda i: (i + nblk_off, 0)),
        pl.BlockSpec((blk, H), lambda i: (i, 0)),
        pl.BlockSpec((H, H), full),
        pl.BlockSpec((H, H), full),
        pl.BlockSpec((1, H), full),
        pl.BlockSpec((1, H), full),
        pl.BlockSpec((1, H), full),
    ]
    args = [e, g_packed, W_e, W_out, b_out, gamma_e, beta_e]
    aliases = {}
    if prev is not None:
        in_specs.append(pl.BlockSpec(memory_space=pl.ANY))
        args.append(prev)
        aliases = {7: 0}
    return pl.pallas_call(
        body,
        grid=(grid,),
        in_specs=in_specs,
        out_specs=pl.BlockSpec((blk, H), lambda i: (i + nblk_off, 0)),
        out_shape=jax.ShapeDtypeStruct((E, H), jnp.float32),
        input_output_aliases=aliases,
    )(*args)


def _sc_scatter(e_new, dst2, zrows):
    mesh = plsc.VectorSubcoreMesh(**_MESH)

    @functools.partial(
        pl.kernel,
        out_type=jax.ShapeDtypeStruct((NC, NPAD, H), jnp.float32),
        mesh=mesh,
        scratch_types=[
            pltpu.VMEM((CPW, C), jnp.int32),
            pltpu.VMEM((2, C, H), jnp.float32),
            pltpu.VMEM_SHARED((NPAD, H), jnp.float32),
            pltpu.SemaphoreType.DMA((2,)),
        ],
    )
    def k(e_hbm, d_hbm, z_hbm, out_hbm, di_v, buf, acc, lsem):
        cid = lax.axis_index("c")
        sid = lax.axis_index("s")
        wid = sid * NC + cid
        row0 = sid * RPS
        ebase = wid * EPW

        def load_start(i, b):
            pltpu.async_copy(e_hbm.at[pl.ds(ebase + i * C, C)], buf.at[b],
                             lsem.at[b])

        def load_wait(b):
            pltpu.make_async_copy(e_hbm.at[pl.ds(ebase, C)], buf.at[b],
                                  lsem.at[b]).wait()

        load_start(0, 0)
        pltpu.sync_copy(z_hbm.at[pl.ds(row0, RPS)], acc.at[pl.ds(row0, RPS)])
        pltpu.sync_copy(d_hbm.at[wid], di_v)
        plsc.subcore_barrier()

        def body(i, carry):
            b = lax.rem(i, 2)
            load_wait(b)

            @pl.when(i + 1 < CPW)
            def _():
                load_start(i + 1, 1 - b)

            pltpu.sync_copy(buf.at[b], acc.at[di_v.at[i]], add=True)
            return carry

        lax.fori_loop(0, CPW, body, 0)
        plsc.subcore_barrier()
        pltpu.sync_copy(acc.at[pl.ds(row0, RPS)], out_hbm.at[cid, pl.ds(row0, RPS)])

    return k(e_new, dst2, zrows)


def _tc_node(h_pad, P, W_n1, b_n1, W_n2, b_n2, gamma_n, beta_n):
    blk = 1024
    full = lambda i: (0, 0)

    def body(h_ref, p0_ref, p1_ref, w1_ref, b1_ref, w2_ref, b2_ref, g_ref, b_ref,
             out_ref):
        hv = h_ref[...]
        agg = p0_ref[0] + p1_ref[0]
        w1 = w1_ref[...]
        x = (
            jnp.dot(hv, w1[:H], preferred_element_type=jnp.float32)
            + jnp.dot(agg, w1[H:], preferred_element_type=jnp.float32)
            + b1_ref[...]
        )
        x = x * jax.nn.sigmoid(x)
        hn = hv + jnp.dot(x, w2_ref[...], preferred_element_type=jnp.float32) + b2_ref[...]
        m = jnp.mean(hn, axis=-1, keepdims=True)
        v = jnp.mean((hn - m) ** 2, axis=-1, keepdims=True)
        out_ref[...] = (hn - m) * lax.rsqrt(v + 1e-5) * g_ref[...] + b_ref[...]

    return pl.pallas_call(
        body,
        grid=(NPAD // blk,),
        in_specs=[
            pl.BlockSpec((blk, H), lambda i: (i, 0)),
            pl.BlockSpec((1, blk, H), lambda i: (0, i, 0)),
            pl.BlockSpec((1, blk, H), lambda i: (1, i, 0)),
            pl.BlockSpec((2 * H, H), full),
            pl.BlockSpec((1, H), full),
            pl.BlockSpec((H, H), full),
            pl.BlockSpec((1, H), full),
            pl.BlockSpec((1, H), full),
            pl.BlockSpec((1, H), full),
        ],
        out_specs=pl.BlockSpec((blk, H), lambda i: (i, 0)),
        out_shape=jax.ShapeDtypeStruct((NPAD, H), jnp.float32),
    )(h_pad, P, P, W_n1, b_n1, W_n2, b_n2, gamma_n, beta_n)


def kernel(h, e, edge_index, W_src, b_src, W_dst, W_e, W_out, b_out, W_n1, b_n1,
           W_n2, b_n2, gamma_e, beta_e, gamma_n, beta_n):
    h_pad = jnp.zeros((NPAD, H), jnp.float32).at[:N].set(h)
    Wsb = jnp.stack([W_src, W_dst])
    bsb = jnp.stack([b_src, jnp.zeros_like(b_src)]).reshape(2, 1, H)

    T3 = _tc_transform(h_pad, Wsb, bsb)
    u = lax.bitcast_convert_type(T3.astype(jnp.bfloat16), jnp.uint16).astype(
        jnp.uint32)
    packed = lax.bitcast_convert_type(
        (u[:, :, H // 2:] << 16) | u[:, :, : H // 2], jnp.int32)
    T = jnp.concatenate([packed[0], packed[1]], axis=1)

    E0 = 192000
    src = edge_index[0]
    dst = edge_index[1]
    g0 = _sc_gather(T, src[:E0].reshape(NW, -1, C),
                    dst[:E0].reshape(NW, -1, C), E0, C)
    g1 = _sc_gather(T, src[E0:].reshape(NW, -1, C),
                    dst[E0:].reshape(NW, -1, C), E - E0, C)
    web = W_e.astype(jnp.bfloat16)
    wob = W_out.astype(jnp.bfloat16)
    en0 = _tc_edge(e, g0, None, 0, web, wob, b_out.reshape(1, H),
                   gamma_e.reshape(1, H), beta_e.reshape(1, H))
    e_new = _tc_edge(e, g1, en0, E0 // 8000, web, wob, b_out.reshape(1, H),
                     gamma_e.reshape(1, H), beta_e.reshape(1, H))

    dst2 = edge_index[1].reshape(NW, CPW, C)
    zrows = jnp.zeros((NPAD, H), jnp.float32)
    P = _sc_scatter(e_new, dst2, zrows)

    h_new_pad = _tc_node(h_pad, P, W_n1, b_n1.reshape(1, H), W_n2,
                         b_n2.reshape(1, H), gamma_n.reshape(1, H),
                         beta_n.reshape(1, H))
    return h_new_pad[:N], e_new


# speedup vs baseline: 1.1434x; 1.0207x over previous
import functools

import jax
import jax.numpy as jnp
from jax import lax
from jax.experimental import pallas as pl
from jax.experimental.pallas import tpu as pltpu
from jax.experimental.pallas import tpu_sc as plsc

N = 10000
E = 320000
H = 128

NC = 2
NS = 16
NW = NC * NS

NPAD = 10240
RPS = NPAD // NS
C = 80
EPW = E // NW
CPW = EPW // C

_MESH = dict(core_axis_name="c", subcore_axis_name="s", num_cores=NC,
             num_subcores=NS)


def _tc_transform(h_pad, Wsb, bsb):
    blk = 1024

    def body(h_ref, w_ref, b_ref, out_ref):
        out_ref[0] = (
            jnp.dot(h_ref[...], w_ref[0], preferred_element_type=jnp.float32)
            + b_ref[0]
        )

    return pl.pallas_call(
        body,
        grid=(2, NPAD // blk),
        in_specs=[
            pl.BlockSpec((blk, H), lambda j, i: (i, 0)),
            pl.BlockSpec((1, H, H), lambda j, i: (j, 0, 0)),
            pl.BlockSpec((1, 1, H), lambda j, i: (j, 0, 0)),
        ],
        out_specs=pl.BlockSpec((1, blk, H), lambda j, i: (j, i, 0)),
        out_shape=jax.ShapeDtypeStruct((2, NPAD, H), jnp.float32),
    )(h_pad, Wsb, bsb)


def _sc_gather(T, src2, dstp2, ne, cc):
    mesh = plsc.VectorSubcoreMesh(**_MESH)
    HW = H // 2
    epw = ne // NW
    cpw = epw // cc

    @functools.partial(
        pl.kernel,
        out_type=jax.ShapeDtypeStruct((ne, H), jnp.int32),
        mesh=mesh,
        scratch_types=[
            pltpu.VMEM((cpw, cc), jnp.int32),
            pltpu.VMEM((cpw, cc), jnp.int32),
            pltpu.VMEM((2, cc, H), jnp.int32),
            pltpu.VMEM((2, cc, H), jnp.int32),
            pltpu.SemaphoreType.DMA((2,)),
            pltpu.SemaphoreType.DMA((2,)),
            pltpu.SemaphoreType.DMA((2,)),
            pltpu.SemaphoreType.DMA((2,)),
        ],
    )
    def k(t_hbm, s_hbm, d_hbm, gs_hbm, si_v, di_v, bufs, bufd,
          gss, gsd, wss, wsd):
        wid = lax.axis_index("s") * NC + lax.axis_index("c")
        pltpu.sync_copy(s_hbm.at[wid], si_v)
        pltpu.sync_copy(d_hbm.at[wid], di_v)
        ebase = wid * epw

        def gather_start(i, b):
            pltpu.async_copy(t_hbm.at[si_v.at[i]], bufs.at[b], gss.at[b])
            pltpu.async_copy(t_hbm.at[di_v.at[i]], bufd.at[b], gsd.at[b])

        def gather_wait(b):
            pltpu.make_async_copy(t_hbm.at[si_v.at[0]], bufs.at[b], gss.at[b]).wait()
            pltpu.make_async_copy(t_hbm.at[di_v.at[0]], bufd.at[b], gsd.at[b]).wait()

        def wb_start(i, b):
            off = ebase + i * cc
            pltpu.async_copy(bufs.at[b], gs_hbm.at[pl.ds(off, cc)], wss.at[b])

        def wb_wait(b):
            pltpu.make_async_copy(bufs.at[b], gs_hbm.at[pl.ds(ebase, cc)],
                                  wss.at[b]).wait()

        gather_start(0, 0)

        def body(i, carry):
            b = lax.rem(i, 2)
            nb = 1 - b
            gather_wait(b)

            @plsc.parallel_loop(0, cc, step=1, unroll=8)
            def asm(r):
                for kq in range(HW // 16):
                    col = HW + 16 * kq
                    bufs[b, r, pl.ds(col, 16)] = bufd[b, r, pl.ds(col, 16)]

            wb_start(i, b)

            @pl.when(i + 1 < cpw)
            def _():
                @pl.when(i >= 1)
                def _():
                    wb_wait(nb)

                gather_start(i + 1, nb)

            return carry

        lax.fori_loop(0, cpw, body, 0)
        wb_wait(0)
        wb_wait(1)

    return k(T, src2, dstp2)


def _tc_edge(e, g_packed, prev, nblk_off, W_e, W_out, b_out, gamma_e, beta_e):
    blk = 8000
    HW = H // 2
    grid = g_packed.shape[0] // blk

    def body(e_ref, gp_ref, we_ref, wo_ref, bo_ref, g_ref, b_ref, *rest):
        out_ref = rest[-1]
        M16 = jnp.full((), -65536, jnp.int32)
        ev = e_ref[...]
        gp = gp_ref[...]
        ws = gp[:, :HW]
        wd = gp[:, HW:]
        lo = (lax.bitcast_convert_type(lax.shift_left(ws, 16), jnp.float32)
              + lax.bitcast_convert_type(lax.shift_left(wd, 16), jnp.float32))
        hi = (lax.bitcast_convert_type(ws & M16, jnp.float32)
              + lax.bitcast_convert_type(wd & M16, jnp.float32))
        z = jnp.concatenate([lo, hi], axis=1) + jnp.dot(
            ev.astype(jnp.bfloat16), we_ref[...],
            preferred_element_type=jnp.float32,
        )
        z = z * jax.nn.sigmoid(z)
        en = ev + jnp.dot(z.astype(jnp.bfloat16), wo_ref[...],
                          preferred_element_type=jnp.float32) + bo_ref[...]
        m = jnp.mean(en, axis=-1, keepdims=True)
        v = jnp.mean((en - m) ** 2, axis=-1, keepdims=True)
        out_ref[...] = (en - m) * lax.rsqrt(v + 1e-5) * g_ref[...] + b_ref[...]

    full = lambda i: (0, 0)
    in_specs = [
        pl.BlockSpec((blk, H), lambda i: (i + nblk_off, 0)),
        pl.BlockSpec((blk, H), lambda i: (i, 0)),
        pl.BlockSpec((H, H), full),
        pl.BlockSpec((H, H), full),
        pl.BlockSpec((1, H), full),
        pl.BlockSpec((1, H), full),
        pl.BlockSpec((1, H), full),
    ]
    args = [e, g_packed, W_e, W_out, b_out, gamma_e, beta_e]
    aliases = {}
    if prev is not None:
        in_specs.append(pl.BlockSpec(memory_space=pl.ANY))
        args.append(prev)
        aliases = {7: 0}
    return pl.pallas_call(
        body,
        grid=(grid,),
        in_specs=in_specs,
        out_specs=pl.BlockSpec((blk, H), lambda i: (i + nblk_off, 0)),
        out_shape=jax.ShapeDtypeStruct((E, H), jnp.float32),
        input_output_aliases=aliases,
    )(*args)


def _sc_scatter(e_new, dst2, zrows):
    mesh = plsc.VectorSubcoreMesh(**_MESH)

    @functools.partial(
        pl.kernel,
        out_type=jax.ShapeDtypeStruct((NC, NPAD, H), jnp.float32),
        mesh=mesh,
        scratch_types=[
            pltpu.VMEM((CPW, C), jnp.int32),
            pltpu.VMEM((2, C, H), jnp.float32),
            pltpu.VMEM_SHARED((NPAD, H), jnp.float32),
            pltpu.SemaphoreType.DMA((2,)),
        ],
    )
    def k(e_hbm, d_hbm, z_hbm, out_hbm, di_v, buf, acc, lsem):
        cid = lax.axis_index("c")
        sid = lax.axis_index("s")
        wid = sid * NC + cid
        row0 = sid * RPS
        ebase = wid * EPW

        def load_start(i, b):
            pltpu.async_copy(e_hbm.at[pl.ds(ebase + i * C, C)], buf.at[b],
                             lsem.at[b])

        def load_wait(b):
            pltpu.make_async_copy(e_hbm.at[pl.ds(ebase, C)], buf.at[b],
                                  lsem.at[b]).wait()

        load_start(0, 0)
        pltpu.sync_copy(z_hbm.at[pl.ds(row0, RPS)], acc.at[pl.ds(row0, RPS)])
        pltpu.sync_copy(d_hbm.at[wid], di_v)
        plsc.subcore_barrier()

        def body(i, carry):
            b = lax.rem(i, 2)
            load_wait(b)

            @pl.when(i + 1 < CPW)
            def _():
                load_start(i + 1, 1 - b)

            pltpu.sync_copy(buf.at[b], acc.at[di_v.at[i]], add=True)
            return carry

        lax.fori_loop(0, CPW, body, 0)
        plsc.subcore_barrier()
        pltpu.sync_copy(acc.at[pl.ds(row0, RPS)], out_hbm.at[cid, pl.ds(row0, RPS)])

    return k(e_new, dst2, zrows)


def _tc_node(h_pad, P, W_n1, b_n1, W_n2, b_n2, gamma_n, beta_n):
    blk = 1024
    full = lambda i: (0, 0)

    def body(h_ref, p0_ref, p1_ref, w1_ref, b1_ref, w2_ref, b2_ref, g_ref, b_ref,
             out_ref):
        hv = h_ref[...]
        agg = p0_ref[0] + p1_ref[0]
        w1 = w1_ref[...]
        x = (
            jnp.dot(hv, w1[:H], preferred_element_type=jnp.float32)
            + jnp.dot(agg, w1[H:], preferred_element_type=jnp.float32)
            + b1_ref[...]
        )
        x = x * jax.nn.sigmoid(x)
        hn = hv + jnp.dot(x, w2_ref[...], preferred_element_type=jnp.float32) + b2_ref[...]
        m = jnp.mean(hn, axis=-1, keepdims=True)
        v = jnp.mean((hn - m) ** 2, axis=-1, keepdims=True)
        out_ref[...] = (hn - m) * lax.rsqrt(v + 1e-5) * g_ref[...] + b_ref[...]

    return pl.pallas_call(
        body,
        grid=(NPAD // blk,),
        in_specs=[
            pl.BlockSpec((blk, H), lambda i: (i, 0)),
            pl.BlockSpec((1, blk, H), lambda i: (0, i, 0)),
            pl.BlockSpec((1, blk, H), lambda i: (1, i, 0)),
            pl.BlockSpec((2 * H, H), full),
            pl.BlockSpec((1, H), full),
            pl.BlockSpec((H, H), full),
            pl.BlockSpec((1, H), full),
            pl.BlockSpec((1, H), full),
            pl.BlockSpec((1, H), full),
        ],
        out_specs=pl.BlockSpec((blk, H), lambda i: (i, 0)),
        out_shape=jax.ShapeDtypeStruct((NPAD, H), jnp.float32),
    )(h_pad, P, P, W_n1, b_n1, W_n2, b_n2, gamma_n, beta_n)


def kernel(h, e, edge_index, W_src, b_src, W_dst, W_e, W_out, b_out, W_n1, b_n1,
           W_n2, b_n2, gamma_e, beta_e, gamma_n, beta_n):
    h_pad = jnp.zeros((NPAD, H), jnp.float32).at[:N].set(h)
    Wsb = jnp.stack([W_src, W_dst])
    bsb = jnp.stack([b_src, jnp.zeros_like(b_src)]).reshape(2, 1, H)

    T3 = _tc_transform(h_pad, Wsb, bsb)
    u = lax.bitcast_convert_type(T3.astype(jnp.bfloat16), jnp.uint16).astype(
        jnp.uint32)
    packed = lax.bitcast_convert_type(
        (u[:, :, H // 2:] << 16) | u[:, :, : H // 2], jnp.int32)
    T = jnp.concatenate([packed[0], packed[1]], axis=1)

    bounds = [0, 128000, 256000, E]
    src = edge_index[0]
    dst = edge_index[1]
    web = W_e.astype(jnp.bfloat16)
    wob = W_out.astype(jnp.bfloat16)
    gs = [
        _sc_gather(T, src[lo:hi].reshape(NW, -1, C),
                   dst[lo:hi].reshape(NW, -1, C), hi - lo, C)
        for lo, hi in zip(bounds[:-1], bounds[1:])
    ]
    e_new = None
    for k, gk in enumerate(gs):
        e_new = _tc_edge(e, gk, e_new, bounds[k] // 8000, web, wob,
                         b_out.reshape(1, H), gamma_e.reshape(1, H),
                         beta_e.reshape(1, H))

    dst2 = edge_index[1].reshape(NW, CPW, C)
    zrows = jnp.zeros((NPAD, H), jnp.float32)
    P = _sc_scatter(e_new, dst2, zrows)

    h_new_pad = _tc_node(h_pad, P, W_n1, b_n1.reshape(1, H), W_n2,
                         b_n2.reshape(1, H), gamma_n.reshape(1, H),
                         beta_n.reshape(1, H))
    return h_new_pad[:N], e_new


# speedup vs baseline: 1.1658x; 1.0196x over previous
import functools

import jax
import jax.numpy as jnp
from jax import lax
from jax.experimental import pallas as pl
from jax.experimental.pallas import tpu as pltpu
from jax.experimental.pallas import tpu_sc as plsc

N = 10000
E = 320000
H = 128

NC = 2
NS = 16
NW = NC * NS

NPAD = 10240
RPS = NPAD // NS
C = 80
EPW = E // NW
CPW = EPW // C

_MESH = dict(core_axis_name="c", subcore_axis_name="s", num_cores=NC,
             num_subcores=NS)


def _tc_transform(h_pad, Wsb, bsb):
    blk = 1024

    def body(h_ref, w_ref, b_ref, out_ref):
        out_ref[0] = (
            jnp.dot(h_ref[...], w_ref[0], preferred_element_type=jnp.float32)
            + b_ref[0]
        )

    return pl.pallas_call(
        body,
        grid=(2, NPAD // blk),
        in_specs=[
            pl.BlockSpec((blk, H), lambda j, i: (i, 0)),
            pl.BlockSpec((1, H, H), lambda j, i: (j, 0, 0)),
            pl.BlockSpec((1, 1, H), lambda j, i: (j, 0, 0)),
        ],
        out_specs=pl.BlockSpec((1, blk, H), lambda j, i: (j, i, 0)),
        out_shape=jax.ShapeDtypeStruct((2, NPAD, H), jnp.float32),
    )(h_pad, Wsb, bsb)


def _sc_gather(T, src2, dstp2, ne, cc):
    mesh = plsc.VectorSubcoreMesh(**_MESH)
    HW = H // 2
    epw = ne // NW
    cpw = epw // cc

    @functools.partial(
        pl.kernel,
        out_type=jax.ShapeDtypeStruct((ne, H), jnp.int32),
        mesh=mesh,
        scratch_types=[
            pltpu.VMEM((cpw, cc), jnp.int32),
            pltpu.VMEM((cpw, cc), jnp.int32),
            pltpu.VMEM((2, cc, H), jnp.int32),
            pltpu.VMEM((2, cc, H), jnp.int32),
            pltpu.SemaphoreType.DMA((2,)),
            pltpu.SemaphoreType.DMA((2,)),
            pltpu.SemaphoreType.DMA((2,)),
            pltpu.SemaphoreType.DMA((2,)),
        ],
    )
    def k(t_hbm, s_hbm, d_hbm, gs_hbm, si_v, di_v, bufs, bufd,
          gss, gsd, wss, wsd):
        wid = lax.axis_index("s") * NC + lax.axis_index("c")
        pltpu.sync_copy(s_hbm.at[wid], si_v)
        pltpu.sync_copy(d_hbm.at[wid], di_v)
        ebase = wid * epw

        def gather_start(i, b):
            pltpu.async_copy(t_hbm.at[si_v.at[i]], bufs.at[b], gss.at[b])
            pltpu.async_copy(t_hbm.at[di_v.at[i]], bufd.at[b], gsd.at[b])

        def gather_wait(b):
            pltpu.make_async_copy(t_hbm.at[si_v.at[0]], bufs.at[b], gss.at[b]).wait()
            pltpu.make_async_copy(t_hbm.at[di_v.at[0]], bufd.at[b], gsd.at[b]).wait()

        def wb_start(i, b):
            off = ebase + i * cc
            pltpu.async_copy(bufs.at[b], gs_hbm.at[pl.ds(off, cc)], wss.at[b])

        def wb_wait(b):
            pltpu.make_async_copy(bufs.at[b], gs_hbm.at[pl.ds(ebase, cc)],
                                  wss.at[b]).wait()

        gather_start(0, 0)

        def body(i, carry):
            b = lax.rem(i, 2)
            nb = 1 - b
            gather_wait(b)

            @plsc.parallel_loop(0, cc, step=1, unroll=8)
            def asm(r):
                for kq in range(HW // 16):
                    col = HW + 16 * kq
                    bufs[b, r, pl.ds(col, 16)] = bufd[b, r, pl.ds(col, 16)]

            wb_start(i, b)

            @pl.when(i + 1 < cpw)
            def _():
                @pl.when(i >= 1)
                def _():
                    wb_wait(nb)

                gather_start(i + 1, nb)

            return carry

        lax.fori_loop(0, cpw, body, 0)
        wb_wait(0)
        wb_wait(1)

    return k(T, src2, dstp2)


def _tc_edge(e, g_packed, prev, nblk_off, W_e, W_out, b_out, gamma_e, beta_e):
    blk = 8000
    HW = H // 2
    grid = g_packed.shape[0] // blk

    def body(e_ref, gp_ref, we_ref, wo_ref, bo_ref, g_ref, b_ref, *rest):
        out_ref = rest[-1]
        M16 = jnp.full((), -65536, jnp.int32)
        ev = e_ref[...]
        gp = gp_ref[...]
        ws = gp[:, :HW]
        wd = gp[:, HW:]
        lo = (lax.bitcast_convert_type(lax.shift_left(ws, 16), jnp.float32)
              + lax.bitcast_convert_type(lax.shift_left(wd, 16), jnp.float32))
        hi = (lax.bitcast_convert_type(ws & M16, jnp.float32)
              + lax.bitcast_convert_type(wd & M16, jnp.float32))
        z = jnp.concatenate([lo, hi], axis=1) + jnp.dot(
            ev.astype(jnp.bfloat16), we_ref[...],
            preferred_element_type=jnp.float32,
        )
        z = z * jax.nn.sigmoid(z)
        en = ev + jnp.dot(z.astype(jnp.bfloat16), wo_ref[...],
                          preferred_element_type=jnp.float32) + bo_ref[...]
        m = jnp.mean(en, axis=-1, keepdims=True)
        v = jnp.mean((en - m) ** 2, axis=-1, keepdims=True)
        out_ref[...] = (en - m) * lax.rsqrt(v + 1e-5) * g_ref[...] + b_ref[...]

    full = lambda i: (0, 0)
    in_specs = [
        pl.BlockSpec((blk, H), lambda i: (i + nblk_off, 0)),
        pl.BlockSpec((blk, H), lambda i: (i, 0)),
        pl.BlockSpec((H, H), full),
        pl.BlockSpec((H, H), full),
        pl.BlockSpec((1, H), full),
        pl.BlockSpec((1, H), full),
        pl.BlockSpec((1, H), full),
    ]
    args = [e, g_packed, W_e, W_out, b_out, gamma_e, beta_e]
    aliases = {}
    if prev is not None:
        in_specs.append(pl.BlockSpec(memory_space=pl.ANY))
        args.append(prev)
        aliases = {7: 0}
    return pl.pallas_call(
        body,
        grid=(grid,),
        in_specs=in_specs,
        out_specs=pl.BlockSpec((blk, H), lambda i: (i + nblk_off, 0)),
        out_shape=jax.ShapeDtypeStruct((E, H), jnp.float32),
        input_output_aliases=aliases,
    )(*args)


def _sc_scatter(e_new, dst2, zrows):
    mesh = plsc.VectorSubcoreMesh(**_MESH)

    @functools.partial(
        pl.kernel,
        out_type=jax.ShapeDtypeStruct((NC, NPAD, H), jnp.float32),
        mesh=mesh,
        scratch_types=[
            pltpu.VMEM((CPW, C), jnp.int32),
            pltpu.VMEM((2, C, H), jnp.float32),
            pltpu.VMEM_SHARED((NPAD, H), jnp.float32),
            pltpu.SemaphoreType.DMA((2,)),
        ],
    )
    def k(e_hbm, d_hbm, z_hbm, out_hbm, di_v, buf, acc, lsem):
        cid = lax.axis_index("c")
        sid = lax.axis_index("s")
        wid = sid * NC + cid
        row0 = sid * RPS
        ebase = wid * EPW

        def load_start(i, b):
            pltpu.async_copy(e_hbm.at[pl.ds(ebase + i * C, C)], buf.at[b],
                             lsem.at[b])

        def load_wait(b):
            pltpu.make_async_copy(e_hbm.at[pl.ds(ebase, C)], buf.at[b],
                                  lsem.at[b]).wait()

        load_start(0, 0)
        pltpu.sync_copy(z_hbm.at[pl.ds(row0, RPS)], acc.at[pl.ds(row0, RPS)])
        pltpu.sync_copy(d_hbm.at[wid], di_v)
        plsc.subcore_barrier()

        def body(i, carry):
            b = lax.rem(i, 2)
            load_wait(b)

            @pl.when(i + 1 < CPW)
            def _():
                load_start(i + 1, 1 - b)

            pltpu.sync_copy(buf.at[b], acc.at[di_v.at[i]], add=True)
            return carry

        lax.fori_loop(0, CPW, body, 0)
        plsc.subcore_barrier()
        pltpu.sync_copy(acc.at[pl.ds(row0, RPS)], out_hbm.at[cid, pl.ds(row0, RPS)])

    return k(e_new, dst2, zrows)


def _tc_node(h_pad, P, W_n1, b_n1, W_n2, b_n2, gamma_n, beta_n):
    blk = 1024
    full = lambda i: (0, 0)

    def body(h_ref, p0_ref, p1_ref, w1_ref, b1_ref, w2_ref, b2_ref, g_ref, b_ref,
             out_ref):
        hv = h_ref[...]
        agg = p0_ref[0] + p1_ref[0]
        w1 = w1_ref[...]
        x = (
            jnp.dot(hv, w1[:H], preferred_element_type=jnp.float32)
            + jnp.dot(agg, w1[H:], preferred_element_type=jnp.float32)
            + b1_ref[...]
        )
        x = x * jax.nn.sigmoid(x)
        hn = hv + jnp.dot(x, w2_ref[...], preferred_element_type=jnp.float32) + b2_ref[...]
        m = jnp.mean(hn, axis=-1, keepdims=True)
        v = jnp.mean((hn - m) ** 2, axis=-1, keepdims=True)
        out_ref[...] = (hn - m) * lax.rsqrt(v + 1e-5) * g_ref[...] + b_ref[...]

    return pl.pallas_call(
        body,
        grid=(NPAD // blk,),
        in_specs=[
            pl.BlockSpec((blk, H), lambda i: (i, 0)),
            pl.BlockSpec((1, blk, H), lambda i: (0, i, 0)),
            pl.BlockSpec((1, blk, H), lambda i: (1, i, 0)),
            pl.BlockSpec((2 * H, H), full),
            pl.BlockSpec((1, H), full),
            pl.BlockSpec((H, H), full),
            pl.BlockSpec((1, H), full),
            pl.BlockSpec((1, H), full),
            pl.BlockSpec((1, H), full),
        ],
        out_specs=pl.BlockSpec((blk, H), lambda i: (i, 0)),
        out_shape=jax.ShapeDtypeStruct((NPAD, H), jnp.float32),
    )(h_pad, P, P, W_n1, b_n1, W_n2, b_n2, gamma_n, beta_n)


def kernel(h, e, edge_index, W_src, b_src, W_dst, W_e, W_out, b_out, W_n1, b_n1,
           W_n2, b_n2, gamma_e, beta_e, gamma_n, beta_n):
    h_pad = jnp.zeros((NPAD, H), jnp.float32).at[:N].set(h)
    Wsb = jnp.stack([W_src, W_dst])
    bsb = jnp.stack([b_src, jnp.zeros_like(b_src)]).reshape(2, 1, H)

    T3 = _tc_transform(h_pad, Wsb, bsb)
    u = lax.bitcast_convert_type(T3.astype(jnp.bfloat16), jnp.uint16).astype(
        jnp.uint32)
    packed = lax.bitcast_convert_type(
        (u[:, :, H // 2:] << 16) | u[:, :, : H // 2], jnp.int32)
    T = jnp.concatenate([packed[0], packed[1]], axis=1)

    bounds = [0, 64000, 128000, 192000, 256000, E]
    src = edge_index[0]
    dst = edge_index[1]
    web = W_e.astype(jnp.bfloat16)
    wob = W_out.astype(jnp.bfloat16)
    gs = [
        _sc_gather(T, src[lo:hi].reshape(NW, -1, C),
                   dst[lo:hi].reshape(NW, -1, C), hi - lo, C)
        for lo, hi in zip(bounds[:-1], bounds[1:])
    ]
    e_new = None
    for k, gk in enumerate(gs):
        e_new = _tc_edge(e, gk, e_new, bounds[k] // 8000, web, wob,
                         b_out.reshape(1, H), gamma_e.reshape(1, H),
                         beta_e.reshape(1, H))

    dst2 = edge_index[1].reshape(NW, CPW, C)
    zrows = jnp.zeros((NPAD, H), jnp.float32)
    P = _sc_scatter(e_new, dst2, zrows)

    h_new_pad = _tc_node(h_pad, P, W_n1, b_n1.reshape(1, H), W_n2,
                         b_n2.reshape(1, H), gamma_n.reshape(1, H),
                         beta_n.reshape(1, H))
    return h_new_pad[:N], e_new


# speedup vs baseline: 1.1670x; 1.0010x over previous
import functools

import jax
import jax.numpy as jnp
from jax import lax
from jax.experimental import pallas as pl
from jax.experimental.pallas import tpu as pltpu
from jax.experimental.pallas import tpu_sc as plsc

N = 10000
E = 320000
H = 128

NC = 2
NS = 16
NW = NC * NS

NPAD = 10240
RPS = NPAD // NS
C = 80
EPW = E // NW
CPW = EPW // C

_MESH = dict(core_axis_name="c", subcore_axis_name="s", num_cores=NC,
             num_subcores=NS)


def _tc_transform(h_pad, Wsb, bsb):
    blk = 1024

    def body(h_ref, w_ref, b_ref, out_ref):
        out_ref[0] = (
            jnp.dot(h_ref[...], w_ref[0], preferred_element_type=jnp.float32)
            + b_ref[0]
        )

    return pl.pallas_call(
        body,
        grid=(2, NPAD // blk),
        in_specs=[
            pl.BlockSpec((blk, H), lambda j, i: (i, 0)),
            pl.BlockSpec((1, H, H), lambda j, i: (j, 0, 0)),
            pl.BlockSpec((1, 1, H), lambda j, i: (j, 0, 0)),
        ],
        out_specs=pl.BlockSpec((1, blk, H), lambda j, i: (j, i, 0)),
        out_shape=jax.ShapeDtypeStruct((2, NPAD, H), jnp.float32),
    )(h_pad, Wsb, bsb)


def _sc_gather(T, src2, dstp2, ne, cc):
    mesh = plsc.VectorSubcoreMesh(**_MESH)
    HW = H // 2
    epw = ne // NW
    cpw = epw // cc

    @functools.partial(
        pl.kernel,
        out_type=jax.ShapeDtypeStruct((ne, H), jnp.int32),
        mesh=mesh,
        scratch_types=[
            pltpu.VMEM((cpw, cc), jnp.int32),
            pltpu.VMEM((cpw, cc), jnp.int32),
            pltpu.VMEM((2, cc, H), jnp.int32),
            pltpu.VMEM((2, cc, H), jnp.int32),
            pltpu.SemaphoreType.DMA((2,)),
            pltpu.SemaphoreType.DMA((2,)),
            pltpu.SemaphoreType.DMA((2,)),
            pltpu.SemaphoreType.DMA((2,)),
        ],
    )
    def k(t_hbm, s_hbm, d_hbm, gs_hbm, si_v, di_v, bufs, bufd,
          gss, gsd, wss, wsd):
        wid = lax.axis_index("s") * NC + lax.axis_index("c")
        pltpu.sync_copy(s_hbm.at[wid], si_v)
        pltpu.sync_copy(d_hbm.at[wid], di_v)
        ebase = wid * epw

        def gather_start(i, b):
            pltpu.async_copy(t_hbm.at[si_v.at[i]], bufs.at[b], gss.at[b])
            pltpu.async_copy(t_hbm.at[di_v.at[i]], bufd.at[b], gsd.at[b])

        def gather_wait(b):
            pltpu.make_async_copy(t_hbm.at[si_v.at[0]], bufs.at[b], gss.at[b]).wait()
            pltpu.make_async_copy(t_hbm.at[di_v.at[0]], bufd.at[b], gsd.at[b]).wait()

        def wb_start(i, b):
            off = ebase + i * cc
            pltpu.async_copy(bufs.at[b], gs_hbm.at[pl.ds(off, cc)], wss.at[b])

        def wb_wait(b):
            pltpu.make_async_copy(bufs.at[b], gs_hbm.at[pl.ds(ebase, cc)],
                                  wss.at[b]).wait()

        gather_start(0, 0)

        def body(i, carry):
            b = lax.rem(i, 2)
            nb = 1 - b
            gather_wait(b)

            @plsc.parallel_loop(0, cc, step=1, unroll=8)
            def asm(r):
                for kq in range(HW // 16):
                    col = HW + 16 * kq
                    bufs[b, r, pl.ds(col, 16)] = bufd[b, r, pl.ds(col, 16)]

            wb_start(i, b)

            @pl.when(i + 1 < cpw)
            def _():
                @pl.when(i >= 1)
                def _():
                    wb_wait(nb)

                gather_start(i + 1, nb)

            return carry

        lax.fori_loop(0, cpw, body, 0)
        wb_wait(0)
        wb_wait(1)

    return k(T, src2, dstp2)


def _tc_edge(e, g_packed, prev, nblk_off, W_e, W_out, b_out, gamma_e, beta_e):
    blk = 8000
    HW = H // 2
    grid = g_packed.shape[0] // blk

    def body(e_ref, gp_ref, we_ref, wo_ref, bo_ref, g_ref, b_ref, *rest):
        out_ref = rest[-1]
        M16 = jnp.full((), -65536, jnp.int32)
        ev = e_ref[...]
        gp = gp_ref[...]
        ws = gp[:, :HW]
        wd = gp[:, HW:]
        lo = (lax.bitcast_convert_type(lax.shift_left(ws, 16), jnp.float32)
              + lax.bitcast_convert_type(lax.shift_left(wd, 16), jnp.float32))
        hi = (lax.bitcast_convert_type(ws & M16, jnp.float32)
              + lax.bitcast_convert_type(wd & M16, jnp.float32))
        z = jnp.concatenate([lo, hi], axis=1) + jnp.dot(
            ev.astype(jnp.bfloat16), we_ref[...],
            preferred_element_type=jnp.float32,
        )
        z = z * jax.nn.sigmoid(z)
        en = ev + jnp.dot(z.astype(jnp.bfloat16), wo_ref[...],
                          preferred_element_type=jnp.float32) + bo_ref[...]
        m = jnp.mean(en, axis=-1, keepdims=True)
        v = jnp.mean((en - m) ** 2, axis=-1, keepdims=True)
        out_ref[...] = (en - m) * lax.rsqrt(v + 1e-5) * g_ref[...] + b_ref[...]

    full = lambda i: (0, 0)
    in_specs = [
        pl.BlockSpec((blk, H), lambda i: (i + nblk_off, 0)),
        pl.BlockSpec((blk, H), lambda i: (i, 0)),
        pl.BlockSpec((H, H), full),
        pl.BlockSpec((H, H), full),
        pl.BlockSpec((1, H), full),
        pl.BlockSpec((1, H), full),
        pl.BlockSpec((1, H), full),
    ]
    args = [e, g_packed, W_e, W_out, b_out, gamma_e, beta_e]
    aliases = {}
    if prev is not None:
        in_specs.append(pl.BlockSpec(memory_space=pl.ANY))
        args.append(prev)
        aliases = {7: 0}
    return pl.pallas_call(
        body,
        grid=(grid,),
        in_specs=in_specs,
        out_specs=pl.BlockSpec((blk, H), lambda i: (i + nblk_off, 0)),
        out_shape=jax.ShapeDtypeStruct((E, H), jnp.float32),
        input_output_aliases=aliases,
    )(*args)


def _sc_scatter(e_new, dst2, zrows):
    mesh = plsc.VectorSubcoreMesh(**_MESH)

    @functools.partial(
        pl.kernel,
        out_type=jax.ShapeDtypeStruct((NC, NPAD, H), jnp.float32),
        mesh=mesh,
        scratch_types=[
            pltpu.VMEM((CPW, C), jnp.int32),
            pltpu.VMEM((2, C, H), jnp.float32),
            pltpu.VMEM_SHARED((NPAD, H), jnp.float32),
            pltpu.SemaphoreType.DMA((2,)),
        ],
    )
    def k(e_hbm, d_hbm, z_hbm, out_hbm, di_v, buf, acc, lsem):
        cid = lax.axis_index("c")
        sid = lax.axis_index("s")
        wid = sid * NC + cid
        row0 = sid * RPS
        ebase = wid * EPW

        def load_start(i, b):
            pltpu.async_copy(e_hbm.at[pl.ds(ebase + i * C, C)], buf.at[b],
                             lsem.at[b])

        def load_wait(b):
            pltpu.make_async_copy(e_hbm.at[pl.ds(ebase, C)], buf.at[b],
                                  lsem.at[b]).wait()

        load_start(0, 0)
        pltpu.sync_copy(z_hbm.at[pl.ds(row0, RPS)], acc.at[pl.ds(row0, RPS)])
        pltpu.sync_copy(d_hbm.at[wid], di_v)
        plsc.subcore_barrier()

        def body(i, carry):
            b = lax.rem(i, 2)
            load_wait(b)

            @pl.when(i + 1 < CPW)
            def _():
                load_start(i + 1, 1 - b)

            pltpu.sync_copy(buf.at[b], acc.at[di_v.at[i]], add=True)
            return carry

        lax.fori_loop(0, CPW, body, 0)
        plsc.subcore_barrier()
        pltpu.sync_copy(acc.at[pl.ds(row0, RPS)], out_hbm.at[cid, pl.ds(row0, RPS)])

    return k(e_new, dst2, zrows)


def _tc_node(h_pad, P, W_n1, b_n1, W_n2, b_n2, gamma_n, beta_n):
    blk = 1024
    full = lambda i: (0, 0)

    def body(h_ref, p0_ref, p1_ref, w1_ref, b1_ref, w2_ref, b2_ref, g_ref, b_ref,
             out_ref):
        hv = h_ref[...]
        agg = p0_ref[0] + p1_ref[0]
        w1 = w1_ref[...]
        x = (
            jnp.dot(hv, w1[:H], preferred_element_type=jnp.float32)
            + jnp.dot(agg, w1[H:], preferred_element_type=jnp.float32)
            + b1_ref[...]
        )
        x = x * jax.nn.sigmoid(x)
        hn = hv + jnp.dot(x, w2_ref[...], preferred_element_type=jnp.float32) + b2_ref[...]
        m = jnp.mean(hn, axis=-1, keepdims=True)
        v = jnp.mean((hn - m) ** 2, axis=-1, keepdims=True)
        out_ref[...] = (hn - m) * lax.rsqrt(v + 1e-5) * g_ref[...] + b_ref[...]

    return pl.pallas_call(
        body,
        grid=(NPAD // blk,),
        in_specs=[
            pl.BlockSpec((blk, H), lambda i: (i, 0)),
            pl.BlockSpec((1, blk, H), lambda i: (0, i, 0)),
            pl.BlockSpec((1, blk, H), lambda i: (1, i, 0)),
            pl.BlockSpec((2 * H, H), full),
            pl.BlockSpec((1, H), full),
            pl.BlockSpec((H, H), full),
            pl.BlockSpec((1, H), full),
            pl.BlockSpec((1, H), full),
            pl.BlockSpec((1, H), full),
        ],
        out_specs=pl.BlockSpec((blk, H), lambda i: (i, 0)),
        out_shape=jax.ShapeDtypeStruct((NPAD, H), jnp.float32),
    )(h_pad, P, P, W_n1, b_n1, W_n2, b_n2, gamma_n, beta_n)


def kernel(h, e, edge_index, W_src, b_src, W_dst, W_e, W_out, b_out, W_n1, b_n1,
           W_n2, b_n2, gamma_e, beta_e, gamma_n, beta_n):
    h_pad = jnp.zeros((NPAD, H), jnp.float32).at[:N].set(h)
    Wsb = jnp.stack([W_src, W_dst])
    bsb = jnp.stack([b_src, jnp.zeros_like(b_src)]).reshape(2, 1, H)

    T3 = _tc_transform(h_pad, Wsb, bsb)
    u = lax.bitcast_convert_type(T3.astype(jnp.bfloat16), jnp.uint16).astype(
        jnp.uint32)
    packed = lax.bitcast_convert_type(
        (u[:, :, H // 2:] << 16) | u[:, :, : H // 2], jnp.int32)
    T = jnp.concatenate([packed[0], packed[1]], axis=1)

    bounds = list(range(0, E + 1, 64000))
    src = edge_index[0]
    dst = edge_index[1]
    web = W_e.astype(jnp.bfloat16)
    wob = W_out.astype(jnp.bfloat16)
    gs = [
        _sc_gather(T, src[lo:hi].reshape(NW, -1, C),
                   dst[lo:hi].reshape(NW, -1, C), hi - lo, C)
        for lo, hi in zip(bounds[:-1], bounds[1:])
    ]
    e_new = None
    for k, gk in enumerate(gs):
        e_new = _tc_edge(e, gk, e_new, bounds[k] // 8000, web, wob,
                         b_out.reshape(1, H), gamma_e.reshape(1, H),
                         beta_e.reshape(1, H))

    dst2 = edge_index[1].reshape(NW, CPW, C)
    zrows = jnp.zeros((NPAD, H), jnp.float32)
    P = _sc_scatter(e_new, dst2, zrows)

    h_new_pad = _tc_node(h_pad, P, W_n1, b_n1.reshape(1, H), W_n2,
                         b_n2.reshape(1, H), gamma_n.reshape(1, H),
                         beta_n.reshape(1, H))
    return h_new_pad[:N], e_new
